# baseline (speedup 1.0000x reference)
import functools

import jax
import jax.numpy as jnp
from jax import lax
from jax.experimental import pallas as pl
from jax.experimental.pallas import tpu as pltpu
from jax.experimental.pallas import tpu_sc as plsc

VOCAB = 100000
D_MODEL = 64
BATCH = 1024

NUM_CORES = 2
NUM_SUBCORES = 16
NUM_WORKERS = NUM_CORES * NUM_SUBCORES
B_PER_W = BATCH // NUM_WORKERS


@functools.cache
def _make_gather_sc():
    mesh = plsc.VectorSubcoreMesh(core_axis_name="c", subcore_axis_name="s")

    @functools.partial(
        pl.kernel,
        mesh=mesh,
        compiler_params=pltpu.CompilerParams(use_tc_tiling_on_sc=False),
        out_type=jax.ShapeDtypeStruct((BATCH, D_MODEL), jnp.float32),
        scratch_types=[
            pltpu.VMEM((B_PER_W,), jnp.int32),
            pltpu.VMEM((B_PER_W, D_MODEL), jnp.float32),
            pltpu.SemaphoreType.DMA,
        ],
    )
    def gather_rows_sc(table_hbm, idx_hbm, out_hbm, idx_v, rows_v, sem):
        wid = lax.axis_index("s") * NUM_CORES + lax.axis_index("c")
        base = wid * B_PER_W
        pltpu.sync_copy(idx_hbm.at[pl.ds(base, B_PER_W)], idx_v)
        pltpu.async_copy(table_hbm.at[idx_v], rows_v, sem).wait()
        pltpu.sync_copy(rows_v, out_hbm.at[pl.ds(base, B_PER_W)])

    return gather_rows_sc


V_BLK = 4096
N_FULL = VOCAB // V_BLK
N_BLK = N_FULL + 1
TAIL_A = 1664
TAIL_B = 32
N_SPAN = BATCH // 8


def _proj_body(x_ref, w_ref, b_ref, out_hbm, acc_ref, tail_ref, sems, tsem):
    j = pl.program_id(0)
    slot = lax.rem(j, 2)

    @pl.when(jnp.logical_and(j >= 2, j <= N_FULL))
    def _():
        pltpu.make_async_copy(
            acc_ref.at[slot],
            out_hbm.at[:, pl.ds(0, V_BLK)],
            sems.at[slot],
        ).wait()

    x16 = x_ref[...].astype(jnp.bfloat16)
    w16 = w_ref[...].astype(jnp.bfloat16)
    acc = lax.dot_general(
        x16, w16,
        (((1,), (1,)), ((), ())),
        preferred_element_type=jnp.float32,
    )
    acc = acc + b_ref[...]

    @pl.when(j < N_FULL)
    def _():
        acc_ref[slot] = acc
        for k in range(N_SPAN):
            pltpu.make_async_copy(
                acc_ref.at[slot, pl.ds(8 * k, 8), :],
                out_hbm.at[pl.ds(8 * k, 8), pl.ds(j * V_BLK, V_BLK)],
                sems.at[slot],
            ).start()

    @pl.when(j == N_FULL)
    def _():
        acc_ref[slot] = acc
        tail_ref[...] = jax.lax.slice(acc, (0, TAIL_A), (BATCH, TAIL_A + TAIL_B))
        for k in range(N_SPAN):
            pltpu.make_async_copy(
                acc_ref.at[slot, pl.ds(8 * k, 8), pl.ds(0, TAIL_A)],
                out_hbm.at[pl.ds(8 * k, 8), pl.ds(N_FULL * V_BLK, TAIL_A)],
                sems.at[slot],
            ).start()
        pltpu.make_async_copy(
            tail_ref,
            out_hbm.at[:, pl.ds(N_FULL * V_BLK + TAIL_A, TAIL_B)],
            tsem,
        ).start()
        pltpu.make_async_copy(
            acc_ref.at[1 - slot],
            out_hbm.at[:, pl.ds(0, V_BLK)],
            sems.at[1 - slot],
        ).wait()
        pltpu.make_async_copy(
            acc_ref.at[slot, :, pl.ds(0, TAIL_A)],
            out_hbm.at[:, pl.ds(0, TAIL_A)],
            sems.at[slot],
        ).wait()
        pltpu.make_async_copy(
            tail_ref,
            out_hbm.at[:, pl.ds(N_FULL * V_BLK + TAIL_A, TAIL_B)],
            tsem,
        ).wait()


def kernel(input_ids, embed_table, W, b):
    ids = input_ids.astype(jnp.int32)
    x = _make_gather_sc()(embed_table, ids)
    b2 = b.reshape(1, VOCAB)
    out = pl.pallas_call(
        _proj_body,
        grid=(N_BLK,),
        in_specs=[
            pl.BlockSpec((BATCH, D_MODEL), lambda j: (0, 0)),
            pl.BlockSpec((V_BLK, D_MODEL), lambda j: (j, 0)),
            pl.BlockSpec((1, V_BLK), lambda j: (0, j)),
        ],
        out_specs=pl.BlockSpec(memory_space=pl.ANY),
        out_shape=jax.ShapeDtypeStruct((BATCH, VOCAB), jnp.float32),
        scratch_shapes=[
            pltpu.VMEM((2, BATCH, V_BLK), jnp.float32),
            pltpu.VMEM((BATCH, TAIL_B), jnp.float32),
            pltpu.SemaphoreType.DMA((2,)),
            pltpu.SemaphoreType.DMA,
        ],
    )(x, W, b2)
    return out


# speedup vs baseline: 2.0604x; 2.0604x over previous
import functools

import jax
import jax.numpy as jnp
from jax import lax
from jax.experimental import pallas as pl
from jax.experimental.pallas import tpu as pltpu
from jax.experimental.pallas import tpu_sc as plsc

VOCAB = 100000
D_MODEL = 64
BATCH = 1024

NUM_CORES = 2
NUM_SUBCORES = 16
NUM_WORKERS = NUM_CORES * NUM_SUBCORES
B_PER_W = BATCH // NUM_WORKERS


@functools.cache
def _make_gather_sc():
    mesh = plsc.VectorSubcoreMesh(core_axis_name="c", subcore_axis_name="s")

    @functools.partial(
        pl.kernel,
        mesh=mesh,
        compiler_params=pltpu.CompilerParams(use_tc_tiling_on_sc=False),
        out_type=jax.ShapeDtypeStruct((BATCH, D_MODEL), jnp.float32),
        scratch_types=[
            pltpu.VMEM((B_PER_W,), jnp.int32),
            pltpu.VMEM((B_PER_W, D_MODEL), jnp.float32),
            pltpu.SemaphoreType.DMA,
        ],
    )
    def gather_rows_sc(table_hbm, idx_hbm, out_hbm, idx_v, rows_v, sem):
        wid = lax.axis_index("s") * NUM_CORES + lax.axis_index("c")
        base = wid * B_PER_W
        pltpu.sync_copy(idx_hbm.at[pl.ds(base, B_PER_W)], idx_v)
        pltpu.async_copy(table_hbm.at[idx_v], rows_v, sem).wait()
        pltpu.sync_copy(rows_v, out_hbm.at[pl.ds(base, B_PER_W)])

    return gather_rows_sc


V_BLK = 4096
N_BLK = (VOCAB + V_BLK - 1) // V_BLK
D_AUG = D_MODEL + 1


def _proj_body(wa_ref, xa_ref, out_ref):
    out_ref[...] = lax.dot_general(
        wa_ref[...], xa_ref[...],
        (((1,), (1,)), ((), ())),
        preferred_element_type=jnp.float32,
    )


def kernel(input_ids, embed_table, W, b):
    ids = input_ids.astype(jnp.int32)
    x = _make_gather_sc()(embed_table, ids)
    xa = jnp.concatenate(
        [x.astype(jnp.bfloat16), jnp.ones((BATCH, 1), jnp.bfloat16)], axis=1)
    wa = jnp.concatenate(
        [W.astype(jnp.bfloat16), b.astype(jnp.bfloat16)[:, None]], axis=1)
    out_t = pl.pallas_call(
        _proj_body,
        grid=(N_BLK,),
        in_specs=[
            pl.BlockSpec((V_BLK, D_AUG), lambda j: (j, 0)),
            pl.BlockSpec((BATCH, D_AUG), lambda j: (0, 0)),
        ],
        out_specs=pl.BlockSpec((V_BLK, BATCH), lambda j: (j, 0)),
        out_shape=jax.ShapeDtypeStruct((VOCAB, BATCH), jnp.float32),
    )(wa, xa)
    return out_t.T


# speedup vs baseline: 2.8355x; 1.3762x over previous
import functools

import jax
import jax.numpy as jnp
from jax import lax
from jax.experimental import pallas as pl
from jax.experimental.pallas import tpu as pltpu
from jax.experimental.pallas import tpu_sc as plsc

VOCAB = 100000
D_MODEL = 64
BATCH = 1024

NUM_CORES = 2
NUM_SUBCORES = 16
NUM_WORKERS = NUM_CORES * NUM_SUBCORES
B_PER_W = BATCH // NUM_WORKERS


@functools.cache
def _make_gather_sc():
    mesh = plsc.VectorSubcoreMesh(core_axis_name="c", subcore_axis_name="s")

    @functools.partial(
        pl.kernel,
        mesh=mesh,
        compiler_params=pltpu.CompilerParams(use_tc_tiling_on_sc=False),
        out_type=jax.ShapeDtypeStruct((BATCH, D_MODEL), jnp.float32),
        scratch_types=[
            pltpu.VMEM((B_PER_W,), jnp.int32),
            pltpu.VMEM((B_PER_W, D_MODEL), jnp.float32),
            pltpu.SemaphoreType.DMA,
        ],
    )
    def gather_rows_sc(table_hbm, idx_hbm, out_hbm, idx_v, rows_v, sem):
        wid = lax.axis_index("s") * NUM_CORES + lax.axis_index("c")
        base = wid * B_PER_W
        pltpu.sync_copy(idx_hbm.at[pl.ds(base, B_PER_W)], idx_v)
        pltpu.async_copy(table_hbm.at[idx_v], rows_v, sem).wait()
        pltpu.sync_copy(rows_v, out_hbm.at[pl.ds(base, B_PER_W)])

    return gather_rows_sc


V_BLK = 4096
N_BLK = (VOCAB + V_BLK - 1) // V_BLK
D_AUG = D_MODEL + 1


def _proj_body(wt_ref, b_ref, xa_ref, out_ref):
    wa16 = jnp.concatenate(
        [wt_ref[...].astype(jnp.bfloat16), b_ref[...].astype(jnp.bfloat16)],
        axis=0)
    out_ref[...] = lax.dot_general(
        wa16, xa_ref[...],
        (((0,), (1,)), ((), ())),
        preferred_element_type=jnp.float32,
    )


def kernel(input_ids, embed_table, W, b):
    ids = input_ids.astype(jnp.int32)
    x = _make_gather_sc()(embed_table, ids)
    xa = jnp.concatenate(
        [x.astype(jnp.bfloat16), jnp.ones((BATCH, 1), jnp.bfloat16)], axis=1)
    wt = W.T
    b2 = b.reshape(1, VOCAB)
    out_t = pl.pallas_call(
        _proj_body,
        grid=(N_BLK,),
        in_specs=[
            pl.BlockSpec((D_MODEL, V_BLK), lambda j: (0, j)),
            pl.BlockSpec((1, V_BLK), lambda j: (0, j)),
            pl.BlockSpec((BATCH, D_AUG), lambda j: (0, 0)),
        ],
        out_specs=pl.BlockSpec((V_BLK, BATCH), lambda j: (j, 0)),
        out_shape=jax.ShapeDtypeStruct((VOCAB, BATCH), jnp.float32),
    )(wt, b2, xa)
    return out_t.T
